# speedup vs baseline: 1.3154x; 1.3154x over previous
import functools

import jax
import jax.numpy as jnp
from jax import lax
from jax.experimental import pallas as pl
from jax.experimental.pallas import tpu as pltpu
from jax.experimental.pallas import tpu_sc as plsc

_B, _T, _S = 4096, 200, 128
_G = _B * _T
_NW = 32
_GPW = _G // _NW
_ROWS_PW = _GPW // 128
_NC = 2


def _make_sc_gather():
    mesh = plsc.VectorSubcoreMesh(core_axis_name="c", subcore_axis_name="s")

    @functools.partial(
        pl.kernel,
        out_type=jax.ShapeDtypeStruct((_G // 128, 128), jnp.float32),
        mesh=mesh,
        scratch_types=[
            pltpu.VMEM((_ROWS_PW, 128), jnp.int32),
            pltpu.VMEM((_ROWS_PW, 128), jnp.int32),
            pltpu.VMEM((_ROWS_PW, 128), jnp.float32),
            pltpu.SemaphoreType.DMA,
        ],
    )
    def sc_gather(table_hbm, qs_hbm, out_hbm, qs_v, idx_v, val_v, sem):
        wid = lax.axis_index("s") * _NC + lax.axis_index("c")
        row0 = wid * _ROWS_PW

        pltpu.sync_copy(qs_hbm.at[pl.ds(row0, _ROWS_PW)], qs_v)

        iota128 = lax.iota(jnp.int32, 16) * 128

        def compute(j, carry):
            g_base = (row0 + j) * 128
            for o in range(8):
                s_ids = qs_v[j, pl.ds(o * 16, 16)] & (_S - 1)
                base = (g_base + o * 16) * _S
                idx_v[j, pl.ds(o * 16, 16)] = s_ids + (iota128 + base)
            return carry

        lax.fori_loop(0, _ROWS_PW, compute, 0)

        def fire(j, carry):
            pltpu.async_copy(table_hbm.at[idx_v.at[j]], val_v.at[j], sem)
            return carry

        lax.fori_loop(0, _ROWS_PW, fire, 0)

        pltpu.make_async_copy(out_hbm.at[pl.ds(0, _ROWS_PW)], val_v, sem).wait()

        pltpu.sync_copy(val_v, out_hbm.at[pl.ds(row0, _ROWS_PW)])

    return sc_gather


_sc_gather = _make_sc_gather()


def kernel(mastery_levels, question_skills):
    table = jnp.reshape(mastery_levels, (-1,))
    qs2d = jnp.reshape(question_skills.astype(jnp.int32), (_G // 128, 128))
    out2d = _sc_gather(table, qs2d)
    return jnp.reshape(out2d, (_B, _T))


# speedup vs baseline: 1.4704x; 1.1178x over previous
import functools

import jax
import jax.numpy as jnp
from jax import lax
from jax.experimental import pallas as pl
from jax.experimental.pallas import tpu as pltpu
from jax.experimental.pallas import tpu_sc as plsc

_B, _T, _S = 4096, 200, 128
_G = _B * _T
_NW = 32
_GPW = _G // _NW
_ROWS_PW = _GPW // 128
_NC = 2


def _make_sc_gather():
    mesh = plsc.VectorSubcoreMesh(core_axis_name="c", subcore_axis_name="s")

    @functools.partial(
        pl.kernel,
        out_type=jax.ShapeDtypeStruct((_G // 128, 128), jnp.float32),
        mesh=mesh,
        scratch_types=[
            pltpu.VMEM((_ROWS_PW, 128), jnp.int32),
            pltpu.VMEM((_ROWS_PW, 128), jnp.int32),
            pltpu.VMEM((_ROWS_PW, 128), jnp.float32),
            pltpu.SemaphoreType.DMA,
        ],
    )
    def sc_gather(table_hbm, qs_hbm, out_hbm, qs_v, idx_v, val_v, sem):
        wid = lax.axis_index("s") * _NC + lax.axis_index("c")
        row0 = wid * _ROWS_PW

        pltpu.sync_copy(qs_hbm.at[pl.ds(row0, _ROWS_PW)], qs_v)

        base0 = lax.iota(jnp.int32, 16) * 128 + row0 * (128 * 128)

        def compute_and_fire(j, base):
            for o in range(8):
                s_ids = qs_v[j, pl.ds(o * 16, 16)] & (_S - 1)
                idx_v[j, pl.ds(o * 16, 16)] = s_ids + base
                base = base + (16 * 128)
            pltpu.async_copy(table_hbm.at[idx_v.at[j]], val_v.at[j], sem)
            return base

        lax.fori_loop(0, _ROWS_PW, compute_and_fire, base0)

        pltpu.make_async_copy(out_hbm.at[pl.ds(0, _ROWS_PW)], val_v, sem).wait()

        pltpu.sync_copy(val_v, out_hbm.at[pl.ds(row0, _ROWS_PW)])

    return sc_gather


_sc_gather = _make_sc_gather()


def kernel(mastery_levels, question_skills):
    table = jnp.reshape(mastery_levels, (-1,))
    qs2d = jnp.reshape(question_skills.astype(jnp.int32), (_G // 128, 128))
    out2d = _sc_gather(table, qs2d)
    return jnp.reshape(out2d, (_B, _T))


# speedup vs baseline: 1.5141x; 1.0297x over previous
import functools

import jax
import jax.numpy as jnp
from jax import lax
from jax.experimental import pallas as pl
from jax.experimental.pallas import tpu as pltpu
from jax.experimental.pallas import tpu_sc as plsc

_B, _T, _S = 4096, 200, 128
_NW = 32
_RPW = _B // _NW
_EPW = _RPW * _T
_NC = 2
_VC = 12


def _make_sc_gather():
    mesh = plsc.VectorSubcoreMesh(core_axis_name="c", subcore_axis_name="s")

    @functools.partial(
        pl.kernel,
        out_type=jax.ShapeDtypeStruct((_B * _T,), jnp.float32),
        mesh=mesh,
        compiler_params=pltpu.CompilerParams(use_tc_tiling_on_sc=True),
        scratch_types=[
            pltpu.VMEM((_RPW, _T), jnp.int32),
            pltpu.VMEM((_RPW, 16), jnp.int32),
            pltpu.VMEM((_EPW,), jnp.int32),
            pltpu.VMEM((_EPW,), jnp.float32),
            pltpu.SemaphoreType.DMA,
        ],
    )
    def sc_gather(table_hbm, qs_hbm, qst_hbm, out_hbm,
                  qs_v, qst_v, idx_v, val_v, sem):
        wid = lax.axis_index("s") * _NC + lax.axis_index("c")
        row0 = wid * _RPW

        pltpu.sync_copy(qs_hbm.at[pl.ds(row0, _RPW)], qs_v)
        pltpu.sync_copy(qst_hbm.at[pl.ds(row0, _RPW)], qst_v)

        base0 = lax.iota(jnp.int32, 16) * _S + row0 * (_T * _S)

        def compute_and_fire(r, rowbase):
            p = r * _T
            for o in range(_VC):
                s_ids = qs_v[r, pl.ds(o * 16, 16)] & (_S - 1)
                idx_v[pl.ds(p + o * 16, 16)] = s_ids + (rowbase + o * (16 * _S))
            s_ids = qst_v[r, pl.ds(0, 16)] & (_S - 1)
            idx_v[pl.ds(p + _T - 16, 16)] = s_ids + (rowbase + (_T - 16) * _S)
            pltpu.async_copy(table_hbm.at[idx_v.at[pl.ds(p, 128)]],
                             val_v.at[pl.ds(p, 128)], sem)
            pltpu.async_copy(table_hbm.at[idx_v.at[pl.ds(p + 128, _T - 128)]],
                             val_v.at[pl.ds(p + 128, _T - 128)], sem)
            return rowbase + _T * _S

        lax.fori_loop(0, _RPW, compute_and_fire, base0)

        pltpu.make_async_copy(table_hbm.at[pl.ds(0, _EPW)], val_v, sem).wait()

        pltpu.sync_copy(val_v, out_hbm.at[pl.ds(wid * _EPW, _EPW)])

    return sc_gather


_sc_gather = _make_sc_gather()


def kernel(mastery_levels, question_skills):
    table = jnp.reshape(mastery_levels, (-1,))
    qs = question_skills.astype(jnp.int32)
    qs_tail = lax.slice(qs, (0, _T - 16), (_B, _T))
    out_flat = _sc_gather(table, qs, qs_tail)
    return jnp.reshape(out_flat, (_B, _T))


# speedup vs baseline: 1.9830x; 1.3097x over previous
import functools

import jax
import jax.numpy as jnp
from jax import lax
from jax.experimental import pallas as pl
from jax.experimental.pallas import tpu as pltpu
from jax.experimental.pallas import tpu_sc as plsc

_B, _T, _S = 4096, 200, 128
_NW = 32
_CPW = _B // _NW
_NC = 2


def _make_sc_gather():
    mesh = plsc.VectorSubcoreMesh(core_axis_name="c", subcore_axis_name="s")

    @functools.partial(
        pl.kernel,
        out_type=jax.ShapeDtypeStruct((_T, _B), jnp.float32),
        mesh=mesh,
        compiler_params=pltpu.CompilerParams(use_tc_tiling_on_sc=True),
        scratch_types=[
            pltpu.VMEM((_T, _CPW), jnp.int32),
            pltpu.VMEM((_T, _CPW), jnp.int32),
            pltpu.VMEM((_T, _CPW), jnp.float32),
            pltpu.SemaphoreType.DMA,
        ],
    )
    def sc_gather(table_hbm, qst_hbm, out_hbm, qs_v, idx_v, val_v, sem):
        wid = lax.axis_index("s") * _NC + lax.axis_index("c")
        col0 = wid * _CPW

        pltpu.sync_copy(qst_hbm.at[:, pl.ds(col0, _CPW)], qs_v)

        lane_base = lax.iota(jnp.int32, 16) * (_T * _S)

        def compute_and_fire(t, carry):
            for h in range(8):
                s_ids = qs_v[t, pl.ds(h * 16, 16)] & (_S - 1)
                base = (col0 + h * 16) * (_T * _S) + t * _S
                idx_v[t, pl.ds(h * 16, 16)] = s_ids + (lane_base + base)
            pltpu.async_copy(table_hbm.at[idx_v.at[t]], val_v.at[t], sem)
            return carry

        lax.fori_loop(0, _T, compute_and_fire, 0)

        pltpu.make_async_copy(out_hbm.at[:, pl.ds(col0, _CPW)], val_v,
                              sem).wait()

        pltpu.sync_copy(val_v, out_hbm.at[:, pl.ds(col0, _CPW)])

    return sc_gather


_sc_gather = _make_sc_gather()


def kernel(mastery_levels, question_skills):
    table = jnp.reshape(mastery_levels, (-1,))
    qs_t = jnp.transpose(question_skills.astype(jnp.int32))
    out_t = _sc_gather(table, qs_t)
    return jnp.transpose(out_t)
